# baseline (device time: 49117 ns/iter reference)
import jax
import jax.numpy as jnp
from jax import lax
from jax.experimental import pallas as pl
from jax.experimental.pallas import tpu as pltpu

N_DEV = 4
E_PER = 4
N_TOK = 512
D_IN = 256
D_OUT = 512
N_EXP = 16


def kernel(x, router_W, route_idx, expert_W):
    def body(x_ref, rw_ref, idx_ref, ew_ref, out_ref,
             comm_ref, send_sems, recv_sems):
        my_pos = lax.axis_index("i")
        left = lax.rem(my_pos - 1 + N_DEV, N_DEV)
        right = lax.rem(my_pos + 1, N_DEV)

        barrier_sem = pltpu.get_barrier_semaphore()
        for nbr in (left, right):
            pl.semaphore_signal(
                barrier_sem, inc=1,
                device_id=(nbr,), device_id_type=pl.DeviceIdType.MESH,
            )
        pl.semaphore_wait(barrier_sem, 2)

        xv = x_ref[:, :]
        scores = jnp.dot(xv, rw_ref[:, :], preferred_element_type=jnp.float32)
        s_max = jnp.max(scores, axis=1, keepdims=True)
        p = jnp.exp(scores - s_max)
        probs = p / jnp.sum(p, axis=1, keepdims=True)

        e0 = idx_ref[:, 0:1]
        e1 = idx_ref[:, 1:2]
        iota = lax.broadcasted_iota(jnp.int32, (N_TOK, N_EXP), 1)
        g0 = jnp.sum(jnp.where(iota == e0, probs, 0.0), axis=1, keepdims=True)
        g1 = jnp.sum(jnp.where(iota == e1, probs, 0.0), axis=1, keepdims=True)
        gs = g0 + g1

        acc = jnp.zeros((N_TOK, D_OUT), dtype=jnp.float32)
        for le in range(E_PER):
            e_glob = my_pos * E_PER + le
            w = (jnp.where(e0 == e_glob, g0, 0.0)
                 + jnp.where(e1 == e_glob, g1, 0.0)) / gs
            acc = acc + jnp.dot(xv * w, ew_ref[le, :, :],
                                preferred_element_type=jnp.float32)

        out_ref[:, :] = acc
        comm_ref[0, :, :] = acc

        for h in range(N_DEV - 1):
            rdma = pltpu.make_async_remote_copy(
                src_ref=comm_ref.at[h],
                dst_ref=comm_ref.at[h + 1],
                send_sem=send_sems.at[h],
                recv_sem=recv_sems.at[h],
                device_id=(right,),
                device_id_type=pl.DeviceIdType.MESH,
            )
            rdma.start()
            rdma.wait()
            out_ref[:, :] += comm_ref[h + 1, :, :]

    return pl.pallas_call(
        body,
        out_shape=jax.ShapeDtypeStruct((N_TOK, D_OUT), jnp.float32),
        in_specs=[
            pl.BlockSpec(memory_space=pltpu.VMEM),
            pl.BlockSpec(memory_space=pltpu.VMEM),
            pl.BlockSpec(memory_space=pltpu.VMEM),
            pl.BlockSpec(memory_space=pltpu.VMEM),
        ],
        out_specs=pl.BlockSpec(memory_space=pltpu.VMEM),
        scratch_shapes=[
            pltpu.VMEM((N_DEV, N_TOK, D_OUT), jnp.float32),
            pltpu.SemaphoreType.DMA((N_DEV - 1,)),
            pltpu.SemaphoreType.DMA((N_DEV - 1,)),
        ],
        compiler_params=pltpu.CompilerParams(collective_id=0),
    )(x, router_W, route_idx, expert_W)


# device time: 32103 ns/iter; 1.5300x vs baseline; 1.5300x over previous
import jax
import jax.numpy as jnp
from jax import lax
from jax.experimental import pallas as pl
from jax.experimental.pallas import tpu as pltpu

N_DEV = 4
E_PER = 4
N_TOK = 512
D_IN = 256
D_OUT = 512
N_EXP = 16
HALF = N_TOK // 2


def kernel(x, router_W, route_idx, expert_W):
    def body(x_ref, rw_ref, idx_ref, ew_ref, out_ref,
             cw_ref, ccw_ref, s_cw, r_cw, s_ccw, r_ccw):
        my_pos = lax.axis_index("i")
        left = lax.rem(my_pos - 1 + N_DEV, N_DEV)
        right = lax.rem(my_pos + 1, N_DEV)

        barrier_sem = pltpu.get_barrier_semaphore()
        for nbr in (left, right):
            pl.semaphore_signal(
                barrier_sem, inc=1,
                device_id=(nbr,), device_id_type=pl.DeviceIdType.MESH,
            )
        pl.semaphore_wait(barrier_sem, 2)

        xv = x_ref[:, :]
        scores = jnp.dot(xv, rw_ref[:, :], preferred_element_type=jnp.float32)
        s_max = jnp.max(scores, axis=1, keepdims=True)
        p = jnp.exp(scores - s_max)
        probs = p / jnp.sum(p, axis=1, keepdims=True)

        e0 = idx_ref[:, 0:1]
        e1 = idx_ref[:, 1:2]
        iota = lax.broadcasted_iota(jnp.int32, (N_TOK, N_EXP), 1)
        g0 = jnp.sum(jnp.where(iota == e0, probs, 0.0), axis=1, keepdims=True)
        g1 = jnp.sum(jnp.where(iota == e1, probs, 0.0), axis=1, keepdims=True)
        gs = g0 + g1

        gated = []
        for le in range(E_PER):
            e_glob = my_pos * E_PER + le
            w = (jnp.where(e0 == e_glob, g0, 0.0)
                 + jnp.where(e1 == e_glob, g1, 0.0)) / gs
            gated.append(xv * w)
        xg = jnp.concatenate(gated, axis=1)
        ew = ew_ref[:, :, :].reshape(E_PER * D_IN, D_OUT)
        acc = jnp.dot(xg, ew, preferred_element_type=jnp.float32)

        out_ref[:, :] = acc
        cw_ref[0, :, :] = acc[:HALF, :]
        ccw_ref[0, :, :] = acc[HALF:, :]

        def hop_rdmas(h):
            cw = pltpu.make_async_remote_copy(
                src_ref=cw_ref.at[h], dst_ref=cw_ref.at[h + 1],
                send_sem=s_cw.at[h], recv_sem=r_cw.at[h],
                device_id=(right,), device_id_type=pl.DeviceIdType.MESH,
            )
            ccw = pltpu.make_async_remote_copy(
                src_ref=ccw_ref.at[h], dst_ref=ccw_ref.at[h + 1],
                send_sem=s_ccw.at[h], recv_sem=r_ccw.at[h],
                device_id=(left,), device_id_type=pl.DeviceIdType.MESH,
            )
            return cw, ccw

        hops = [hop_rdmas(h) for h in range(N_DEV - 1)]
        hops[0][0].start()
        hops[0][1].start()
        for h in range(N_DEV - 1):
            hops[h][0].wait_recv()
            hops[h][1].wait_recv()
            if h + 1 < N_DEV - 1:
                hops[h + 1][0].start()
                hops[h + 1][1].start()
            out_ref[:HALF, :] += cw_ref[h + 1, :, :]
            out_ref[HALF:, :] += ccw_ref[h + 1, :, :]

        for h in range(N_DEV - 1):
            hops[h][0].wait_send()
            hops[h][1].wait_send()

    return pl.pallas_call(
        body,
        out_shape=jax.ShapeDtypeStruct((N_TOK, D_OUT), jnp.float32),
        in_specs=[
            pl.BlockSpec(memory_space=pltpu.VMEM),
            pl.BlockSpec(memory_space=pltpu.VMEM),
            pl.BlockSpec(memory_space=pltpu.VMEM),
            pl.BlockSpec(memory_space=pltpu.VMEM),
        ],
        out_specs=pl.BlockSpec(memory_space=pltpu.VMEM),
        scratch_shapes=[
            pltpu.VMEM((N_DEV, HALF, D_OUT), jnp.float32),
            pltpu.VMEM((N_DEV, HALF, D_OUT), jnp.float32),
            pltpu.SemaphoreType.DMA((N_DEV - 1,)),
            pltpu.SemaphoreType.DMA((N_DEV - 1,)),
            pltpu.SemaphoreType.DMA((N_DEV - 1,)),
            pltpu.SemaphoreType.DMA((N_DEV - 1,)),
        ],
        compiler_params=pltpu.CompilerParams(collective_id=0),
    )(x, router_W, route_idx, expert_W)


# device time: 25311 ns/iter; 1.9405x vs baseline; 1.2683x over previous
import jax
import jax.numpy as jnp
from jax import lax
from jax.experimental import pallas as pl
from jax.experimental.pallas import tpu as pltpu

N_DEV = 4
E_PER = 4
N_TOK = 512
D_IN = 256
D_OUT = 512
N_EXP = 16
CHUNK = N_TOK // N_DEV


def kernel(x, router_W, route_idx, expert_W):
    def body(x_ref, rw_ref, idx_ref, ew_ref, out_ref,
             rs_buf, red_buf, s_rs, r_rs, s_ag, r_ag):
        me = lax.axis_index("i")

        barrier_sem = pltpu.get_barrier_semaphore()
        for d in range(1, N_DEV):
            pl.semaphore_signal(
                barrier_sem, inc=1,
                device_id=(lax.rem(me + d, N_DEV),),
                device_id_type=pl.DeviceIdType.MESH,
            )
        pl.semaphore_wait(barrier_sem, N_DEV - 1)

        xv = x_ref[:, :]
        scores = jnp.dot(xv, rw_ref[:, :], preferred_element_type=jnp.float32)
        s_max = jnp.max(scores, axis=1, keepdims=True)
        p = jnp.exp(scores - s_max)
        probs = p / jnp.sum(p, axis=1, keepdims=True)

        e0 = idx_ref[:, 0:1]
        e1 = idx_ref[:, 1:2]
        iota = lax.broadcasted_iota(jnp.int32, (N_TOK, N_EXP), 1)
        g0 = jnp.sum(jnp.where(iota == e0, probs, 0.0), axis=1, keepdims=True)
        g1 = jnp.sum(jnp.where(iota == e1, probs, 0.0), axis=1, keepdims=True)
        gs = g0 + g1

        gated = []
        for le in range(E_PER):
            e_glob = me * E_PER + le
            w = (jnp.where(e0 == e_glob, g0, 0.0)
                 + jnp.where(e1 == e_glob, g1, 0.0)) / gs
            gated.append(xv * w)
        xg = jnp.concatenate(gated, axis=1)
        ew = ew_ref[:, :, :].reshape(E_PER * D_IN, D_OUT)
        acc = jnp.dot(xg, ew, preferred_element_type=jnp.float32)
        out_ref[:, :] = acc

        rs_sends = []
        for d in range(1, N_DEV):
            q = lax.rem(me + d, N_DEV)
            k = N_DEV - d
            rdma = pltpu.make_async_remote_copy(
                src_ref=out_ref.at[pl.ds(q * CHUNK, CHUNK)],
                dst_ref=rs_buf.at[k],
                send_sem=s_rs.at[d],
                recv_sem=r_rs.at[k],
                device_id=(q,),
                device_id_type=pl.DeviceIdType.MESH,
            )
            rdma.start()
            rs_sends.append(rdma)
        for rdma in rs_sends:
            rdma.wait_recv()

        red = out_ref[pl.ds(me * CHUNK, CHUNK), :]
        for k in range(1, N_DEV):
            red = red + rs_buf[k, :, :]
        out_ref[pl.ds(me * CHUNK, CHUNK), :] = red
        red_buf[:, :] = red

        ag_sends = []
        for d in range(1, N_DEV):
            q = lax.rem(me + d, N_DEV)
            k = N_DEV - d
            rdma = pltpu.make_async_remote_copy(
                src_ref=red_buf,
                dst_ref=out_ref.at[pl.ds(me * CHUNK, CHUNK)],
                send_sem=s_ag.at[d],
                recv_sem=r_ag.at[k],
                device_id=(q,),
                device_id_type=pl.DeviceIdType.MESH,
            )
            rdma.start()
            ag_sends.append(rdma)
        for rdma in ag_sends:
            rdma.wait_recv()

        for rdma in rs_sends + ag_sends:
            rdma.wait_send()

    return pl.pallas_call(
        body,
        out_shape=jax.ShapeDtypeStruct((N_TOK, D_OUT), jnp.float32),
        in_specs=[
            pl.BlockSpec(memory_space=pltpu.VMEM),
            pl.BlockSpec(memory_space=pltpu.VMEM),
            pl.BlockSpec(memory_space=pltpu.VMEM),
            pl.BlockSpec(memory_space=pltpu.VMEM),
        ],
        out_specs=pl.BlockSpec(memory_space=pltpu.VMEM),
        scratch_shapes=[
            pltpu.VMEM((N_DEV, CHUNK, D_OUT), jnp.float32),
            pltpu.VMEM((CHUNK, D_OUT), jnp.float32),
            pltpu.SemaphoreType.DMA((N_DEV,)),
            pltpu.SemaphoreType.DMA((N_DEV,)),
            pltpu.SemaphoreType.DMA((N_DEV,)),
            pltpu.SemaphoreType.DMA((N_DEV,)),
        ],
        compiler_params=pltpu.CompilerParams(collective_id=0),
    )(x, router_W, route_idx, expert_W)


# device time: 19951 ns/iter; 2.4619x vs baseline; 1.2687x over previous
import jax
import jax.numpy as jnp
from jax import lax
from jax.experimental import pallas as pl
from jax.experimental.pallas import tpu as pltpu

N_DEV = 4
E_PER = 4
N_TOK = 512
D_IN = 256
D_OUT = 512
N_EXP = 16
CHUNK = N_TOK // N_DEV

SEND_ORDER = (2, 3, 1)


def kernel(x, router_W, route_idx, expert_W):
    def body(x_ref, rw_ref, idx_ref, ew_ref, out_ref,
             xg_ref, stage, rs_buf, red_buf, ag_buf,
             s_rs, r_rs, s_ag, r_ag):
        me = lax.axis_index("i")

        barrier_sem = pltpu.get_barrier_semaphore()
        for d in range(1, N_DEV):
            pl.semaphore_signal(
                barrier_sem, inc=1,
                device_id=(lax.rem(me + d, N_DEV),),
                device_id_type=pl.DeviceIdType.MESH,
            )
        pl.semaphore_wait(barrier_sem, N_DEV - 1)

        xv = x_ref[:, :]
        scores = jnp.dot(xv, rw_ref[:, :], preferred_element_type=jnp.float32)
        s_max = jnp.max(scores, axis=1, keepdims=True)
        p = jnp.exp(scores - s_max)
        probs = p / jnp.sum(p, axis=1, keepdims=True)

        e0 = idx_ref[:, 0:1]
        e1 = idx_ref[:, 1:2]
        iota = lax.broadcasted_iota(jnp.int32, (N_TOK, N_EXP), 1)
        g0 = jnp.sum(jnp.where(iota == e0, probs, 0.0), axis=1, keepdims=True)
        g1 = jnp.sum(jnp.where(iota == e1, probs, 0.0), axis=1, keepdims=True)
        gs = g0 + g1

        gated = []
        for le in range(E_PER):
            e_glob = me * E_PER + le
            w = (jnp.where(e0 == e_glob, g0, 0.0)
                 + jnp.where(e1 == e_glob, g1, 0.0)) / gs
            gated.append(xv * w)
        xg_ref[:, :] = jnp.concatenate(gated, axis=1).astype(jnp.bfloat16)
        ew = ew_ref[:, :, :].reshape(E_PER * D_IN, D_OUT).astype(jnp.bfloat16)

        rs_sends = []
        for d in SEND_ORDER:
            q = lax.rem(me + d, N_DEV)
            ck = jnp.dot(xg_ref[pl.ds(q * CHUNK, CHUNK), :], ew,
                         preferred_element_type=jnp.float32)
            stage[d, :, :] = ck.astype(jnp.bfloat16)
            rdma = pltpu.make_async_remote_copy(
                src_ref=stage.at[d],
                dst_ref=rs_buf.at[N_DEV - d],
                send_sem=s_rs.at[d],
                recv_sem=r_rs.at[N_DEV - d],
                device_id=(q,),
                device_id_type=pl.DeviceIdType.MESH,
            )
            rdma.start()
            rs_sends.append(rdma)

        mine = jnp.dot(xg_ref[pl.ds(me * CHUNK, CHUNK), :], ew,
                       preferred_element_type=jnp.float32)
        for rdma in rs_sends:
            rdma.wait_recv()
        red = mine
        for k in range(1, N_DEV):
            red = red + rs_buf[k, :, :].astype(jnp.float32)
        out_ref[pl.ds(me * CHUNK, CHUNK), :] = red
        red_buf[:, :] = red.astype(jnp.bfloat16)

        ag_sends = []
        for d in range(1, N_DEV):
            q = lax.rem(me + d, N_DEV)
            rdma = pltpu.make_async_remote_copy(
                src_ref=red_buf,
                dst_ref=ag_buf.at[N_DEV - d],
                send_sem=s_ag.at[d],
                recv_sem=r_ag.at[N_DEV - d],
                device_id=(q,),
                device_id_type=pl.DeviceIdType.MESH,
            )
            rdma.start()
            ag_sends.append(rdma)
        for rdma in ag_sends:
            rdma.wait_recv()
        for k in range(1, N_DEV):
            src = lax.rem(me + k, N_DEV)
            out_ref[pl.ds(src * CHUNK, CHUNK), :] = (
                ag_buf[k, :, :].astype(jnp.float32))

        for rdma in rs_sends + ag_sends:
            rdma.wait_send()

    return pl.pallas_call(
        body,
        out_shape=jax.ShapeDtypeStruct((N_TOK, D_OUT), jnp.float32),
        in_specs=[
            pl.BlockSpec(memory_space=pltpu.VMEM),
            pl.BlockSpec(memory_space=pltpu.VMEM),
            pl.BlockSpec(memory_space=pltpu.VMEM),
            pl.BlockSpec(memory_space=pltpu.VMEM),
        ],
        out_specs=pl.BlockSpec(memory_space=pltpu.VMEM),
        scratch_shapes=[
            pltpu.VMEM((N_TOK, E_PER * D_IN), jnp.bfloat16),
            pltpu.VMEM((N_DEV, CHUNK, D_OUT), jnp.bfloat16),
            pltpu.VMEM((N_DEV, CHUNK, D_OUT), jnp.bfloat16),
            pltpu.VMEM((CHUNK, D_OUT), jnp.bfloat16),
            pltpu.VMEM((N_DEV, CHUNK, D_OUT), jnp.bfloat16),
            pltpu.SemaphoreType.DMA((N_DEV,)),
            pltpu.SemaphoreType.DMA((N_DEV,)),
            pltpu.SemaphoreType.DMA((N_DEV,)),
            pltpu.SemaphoreType.DMA((N_DEV,)),
        ],
        compiler_params=pltpu.CompilerParams(collective_id=0),
    )(x, router_W, route_idx, expert_W)
